# Initial kernel scaffold; baseline (speedup 1.0000x reference)
import functools

import jax
import jax.numpy as jnp
from jax import lax
from jax.experimental import pallas as pl
from jax.experimental.pallas import tpu as pltpu
from jax.experimental.pallas import tpu_sc as plsc

N_NODES = 10000
N_EDGES = 320000
D = 128

NC = 2
NS = 16
NW = NC * NS
EPT = N_EDGES // NW
CH = 80
NCHUNK = EPT // CH
RPT = N_NODES // NS
ZROWS = 125

_mesh = plsc.VectorSubcoreMesh(core_axis_name="c", subcore_axis_name="s")


@functools.partial(
    pl.kernel,
    out_type=(
        jax.ShapeDtypeStruct((NC, N_NODES, D), jnp.float32),
        jax.ShapeDtypeStruct((NW, N_NODES), jnp.float32),
    ),
    mesh=_mesh,
    scratch_types=[
        pltpu.VMEM((NCHUNK, CH), jnp.int32),
        pltpu.VMEM((NCHUNK, CH), jnp.int32),
        pltpu.VMEM((CH, D), jnp.float32),
        pltpu.VMEM((N_NODES,), jnp.float32),
        pltpu.VMEM((ZROWS, D), jnp.float32),
        pltpu.VMEM_SHARED((N_NODES, D), jnp.float32),
        pltpu.SemaphoreType.DMA,
    ],
)
def _sc_aggregate(src_hbm, dst_hbm, x_hbm, aggp_hbm, degp_hbm,
                  src_v, dst_v, rows_v, deg_v, zero_v, acc_sh, sem):
    c = lax.axis_index("c")
    s = lax.axis_index("s")
    wid = s * NC + c

    pltpu.sync_copy(src_hbm.at[wid], src_v)
    pltpu.sync_copy(dst_hbm.at[wid], dst_v)

    zeros16 = jnp.zeros((16,), jnp.float32)

    def _zero_zbuf(i, carry):
        def _inner(j, carry2):
            zero_v[i, pl.ds(j * 16, 16)] = zeros16
            return carry2
        return lax.fori_loop(0, D // 16, _inner, carry)
    lax.fori_loop(0, ZROWS, _zero_zbuf, 0)

    def _zero_deg(i, carry):
        deg_v[pl.ds(i * 16, 16)] = zeros16
        return carry
    lax.fori_loop(0, N_NODES // 16, _zero_deg, 0)

    for k in range(RPT // ZROWS):
        pltpu.sync_copy(zero_v, acc_sh.at[pl.ds(s * RPT + k * ZROWS, ZROWS)])
    plsc.subcore_barrier()

    ones16 = jnp.ones((16,), jnp.float32)

    def _chunk(ci, carry):
        pltpu.async_copy(x_hbm.at[src_v.at[ci]], rows_v, sem).wait()
        pltpu.sync_copy(rows_v, acc_sh.at[dst_v.at[ci]], add=True)

        def _deg(j, carry2):
            idx = dst_v[ci, pl.ds(j * 16, 16)]
            plsc.addupdate_scatter(deg_v, [idx], ones16)
            return carry2
        return lax.fori_loop(0, CH // 16, _deg, carry)

    lax.fori_loop(0, NCHUNK, _chunk, 0)
    plsc.subcore_barrier()

    pltpu.sync_copy(acc_sh.at[pl.ds(s * RPT, RPT)],
                    aggp_hbm.at[c, pl.ds(s * RPT, RPT)])
    pltpu.sync_copy(deg_v, degp_hbm.at[wid])


def _tc_layer_body(aggp_ref, degp_ref, x_ref, wl_ref, bl_ref, wr_ref, o_ref,
                   *, relu):
    agg = aggp_ref[0] + aggp_ref[1]
    deg = jnp.sum(degp_ref[...], axis=0)
    mean = agg / jnp.maximum(deg, 1.0)[:, None]
    dn = (((1,), (1,)), ((), ()))
    out = lax.dot_general(mean, wl_ref[...], dn,
                          precision=lax.Precision.HIGHEST,
                          preferred_element_type=jnp.float32)
    out = out + bl_ref[...]
    out = out + lax.dot_general(x_ref[...], wr_ref[...], dn,
                                precision=lax.Precision.HIGHEST,
                                preferred_element_type=jnp.float32)
    if relu:
        out = jnp.maximum(out, 0.0)
    o_ref[...] = out


def _tc_layer(aggp, degp, x, wl, bl, wr, relu):
    blk = 1000
    grid = (N_NODES // blk,)
    return pl.pallas_call(
        functools.partial(_tc_layer_body, relu=relu),
        grid=grid,
        in_specs=[
            pl.BlockSpec((NC, blk, D), lambda i: (0, i, 0)),
            pl.BlockSpec((NW, blk), lambda i: (0, i)),
            pl.BlockSpec((blk, D), lambda i: (i, 0)),
            pl.BlockSpec((D, D), lambda i: (0, 0)),
            pl.BlockSpec((1, D), lambda i: (0, 0)),
            pl.BlockSpec((D, D), lambda i: (0, 0)),
        ],
        out_specs=pl.BlockSpec((blk, D), lambda i: (i, 0)),
        out_shape=jax.ShapeDtypeStruct((N_NODES, D), jnp.float32),
    )(aggp, degp, x, wl, bl, wr)


def kernel(x, edge_index, W1l, b1l, W1r, W2l, b2l, W2r):
    src = edge_index[0].reshape(NW, NCHUNK, CH)
    dst = edge_index[1].reshape(NW, NCHUNK, CH)

    aggp1, degp = _sc_aggregate(src, dst, x)
    h = _tc_layer(aggp1, degp, x, W1l, b1l.reshape(1, D), W1r, relu=True)
    aggp2, _ = _sc_aggregate(src, dst, h)
    out = _tc_layer(aggp2, degp, h, W2l, b2l.reshape(1, D), W2r, relu=False)
    return out


# speedup vs baseline: 7.4651x; 7.4651x over previous
import functools

import jax
import jax.numpy as jnp
from jax import lax
from jax.experimental import pallas as pl
from jax.experimental.pallas import tpu as pltpu
from jax.experimental.pallas import tpu_sc as plsc

N_NODES = 10000
N_PAD = 10240
N_EDGES = 320000
D = 128

NC = 2
NS = 16
NW = NC * NS
EPT = N_EDGES // NW
CH = 80
NCHUNK = EPT // CH
SB = 25
NSB = NCHUNK // SB
RPT = N_PAD // NS
ZROWS = 16

_mesh = plsc.VectorSubcoreMesh(core_axis_name="c", subcore_axis_name="s")


@functools.partial(
    pl.kernel,
    out_type=(
        jax.ShapeDtypeStruct((NC, N_PAD, D), jnp.float32),
        jax.ShapeDtypeStruct((NW * N_PAD,), jnp.float32),
    ),
    mesh=_mesh,
    compiler_params=pltpu.CompilerParams(needs_layout_passes=False),
    scratch_types=[
        pltpu.VMEM((SB, CH), jnp.int32),
        pltpu.VMEM((SB, CH), jnp.int32),
        pltpu.VMEM((CH, D), jnp.float32),
        pltpu.VMEM((N_PAD,), jnp.float32),
        pltpu.VMEM((ZROWS, D), jnp.float32),
        pltpu.VMEM_SHARED((N_PAD, D), jnp.float32),
        pltpu.SemaphoreType.DMA,
    ],
)
def _sc_aggregate(src_hbm, dst_hbm, x_hbm, aggp_hbm, degp_hbm,
                  src_v, dst_v, rows_v, deg_v, zero_v, acc_sh, sem):
    c = lax.axis_index("c")
    s = lax.axis_index("s")
    wid = s * NC + c

    zeros16 = jnp.zeros((16,), jnp.float32)

    def _zero_zbuf(i, carry):
        def _inner(j, carry2):
            zero_v[i, pl.ds(j * 16, 16)] = zeros16
            return carry2
        return lax.fori_loop(0, D // 16, _inner, carry)
    lax.fori_loop(0, ZROWS, _zero_zbuf, 0)

    def _zero_deg(i, carry):
        deg_v[pl.ds(i * 16, 16)] = zeros16
        return carry
    lax.fori_loop(0, N_PAD // 16, _zero_deg, 0)

    slab = pl.multiple_of(s * RPT, RPT)

    def _zero_acc(k, carry):
        pltpu.sync_copy(zero_v, acc_sh.at[pl.ds(slab + k * ZROWS, ZROWS)])
        return carry
    lax.fori_loop(0, RPT // ZROWS, _zero_acc, 0)
    plsc.subcore_barrier()

    ones16 = jnp.ones((16,), jnp.float32)

    def _superblock(sb, carry):
        pltpu.sync_copy(src_hbm.at[wid, sb], src_v)
        pltpu.sync_copy(dst_hbm.at[wid, sb], dst_v)

        def _chunk(ci, carry2):
            pltpu.async_copy(x_hbm.at[src_v.at[ci]], rows_v, sem).wait()
            pltpu.sync_copy(rows_v, acc_sh.at[dst_v.at[ci]], add=True)

            def _deg(j, carry3):
                idx = dst_v[ci, pl.ds(j * 16, 16)]
                plsc.addupdate_scatter(deg_v, [idx], ones16)
                return carry3
            return lax.fori_loop(0, CH // 16, _deg, carry2)

        return lax.fori_loop(0, SB, _chunk, carry)

    lax.fori_loop(0, NSB, _superblock, 0)
    plsc.subcore_barrier()

    pltpu.sync_copy(acc_sh.at[pl.ds(slab, RPT)],
                    aggp_hbm.at[c, pl.ds(slab, RPT)])
    pltpu.sync_copy(deg_v, degp_hbm.at[pl.ds(wid * N_PAD, N_PAD)])


def _tc_layer_body(aggp_ref, degp_ref, x_ref, wl_ref, bl_ref, wr_ref, o_ref,
                   *, relu):
    agg = aggp_ref[0] + aggp_ref[1]
    deg = jnp.sum(degp_ref[...], axis=1)
    mean = agg / jnp.maximum(deg, 1.0)[:, None]
    dn = (((1,), (1,)), ((), ()))
    out = lax.dot_general(mean, wl_ref[...], dn,
                          precision=lax.Precision.HIGHEST,
                          preferred_element_type=jnp.float32)
    out = out + bl_ref[...]
    out = out + lax.dot_general(x_ref[...], wr_ref[...], dn,
                                precision=lax.Precision.HIGHEST,
                                preferred_element_type=jnp.float32)
    if relu:
        out = jnp.maximum(out, 0.0)
    o_ref[...] = out


def _tc_layer(aggp, degp_t, x, wl, bl, wr, relu):
    blk = 1024
    grid = (N_PAD // blk,)
    return pl.pallas_call(
        functools.partial(_tc_layer_body, relu=relu),
        grid=grid,
        in_specs=[
            pl.BlockSpec((NC, blk, D), lambda i: (0, i, 0)),
            pl.BlockSpec((blk, NW), lambda i: (i, 0)),
            pl.BlockSpec((blk, D), lambda i: (i, 0)),
            pl.BlockSpec((D, D), lambda i: (0, 0)),
            pl.BlockSpec((1, D), lambda i: (0, 0)),
            pl.BlockSpec((D, D), lambda i: (0, 0)),
        ],
        out_specs=pl.BlockSpec((blk, D), lambda i: (i, 0)),
        out_shape=jax.ShapeDtypeStruct((N_PAD, D), jnp.float32),
    )(aggp, degp_t, x, wl, bl, wr)


def kernel(x, edge_index, W1l, b1l, W1r, W2l, b2l, W2r):
    src = edge_index[0].reshape(NW, NSB, SB, CH)
    dst = edge_index[1].reshape(NW, NSB, SB, CH)
    x_pad = jnp.pad(x, ((0, N_PAD - N_NODES), (0, 0)))

    aggp1, degp = _sc_aggregate(src, dst, x_pad)
    degp_t = degp.reshape(NW, N_PAD).T
    h = _tc_layer(aggp1, degp_t, x_pad, W1l, b1l.reshape(1, D), W1r, relu=True)
    aggp2, _ = _sc_aggregate(src, dst, h)
    out = _tc_layer(aggp2, degp_t, h, W2l, b2l.reshape(1, D), W2r, relu=False)
    return out[:N_NODES]


# speedup vs baseline: 9.2970x; 1.2454x over previous
import functools

import jax
import jax.numpy as jnp
from jax import lax
from jax.experimental import pallas as pl
from jax.experimental.pallas import tpu as pltpu
from jax.experimental.pallas import tpu_sc as plsc

N_NODES = 10000
N_PAD = 10240
N_EDGES = 320000
D = 128

NC = 2
NS = 16
NW = NC * NS
EPT = N_EDGES // NW
CH = 80
NCHUNK = EPT // CH
SB = 25
NSB = NCHUNK // SB
RPT = N_PAD // NS
ZROWS = 8
NPAIR = (SB - 1) // 2

_mesh = plsc.VectorSubcoreMesh(core_axis_name="c", subcore_axis_name="s")


@functools.partial(
    pl.kernel,
    out_type=(
        jax.ShapeDtypeStruct((NC, N_PAD, D), jnp.float32),
        jax.ShapeDtypeStruct((NW * N_PAD,), jnp.float32),
    ),
    mesh=_mesh,
    compiler_params=pltpu.CompilerParams(needs_layout_passes=False),
    scratch_types=[
        pltpu.VMEM((SB, CH), jnp.int32),
        pltpu.VMEM((SB, CH), jnp.int32),
        pltpu.VMEM((CH, D), jnp.float32),
        pltpu.VMEM((CH, D), jnp.float32),
        pltpu.VMEM((N_PAD,), jnp.float32),
        pltpu.VMEM((ZROWS, D), jnp.float32),
        pltpu.VMEM_SHARED((N_PAD, D), jnp.float32),
        pltpu.SemaphoreType.DMA,
        pltpu.SemaphoreType.DMA,
    ],
)
def _sc_aggregate(src_hbm, dst_hbm, x_hbm, aggp_hbm, degp_hbm,
                  src_v, dst_v, rows0_v, rows1_v, deg_v, zero_v, acc_sh,
                  sem0, sem1):
    c = lax.axis_index("c")
    s = lax.axis_index("s")
    wid = s * NC + c

    zeros16 = jnp.zeros((16,), jnp.float32)

    def _zero_zbuf(i, carry):
        def _inner(j, carry2):
            zero_v[i, pl.ds(j * 16, 16)] = zeros16
            return carry2
        return lax.fori_loop(0, D // 16, _inner, carry)
    lax.fori_loop(0, ZROWS, _zero_zbuf, 0)

    def _zero_deg(i, carry):
        deg_v[pl.ds(i * 16, 16)] = zeros16
        return carry
    lax.fori_loop(0, N_PAD // 16, _zero_deg, 0)

    slab = pl.multiple_of(s * RPT, RPT)

    def _zero_acc(k, carry):
        pltpu.sync_copy(zero_v, acc_sh.at[pl.ds(slab + k * ZROWS, ZROWS)])
        return carry
    lax.fori_loop(0, RPT // ZROWS, _zero_acc, 0)
    plsc.subcore_barrier()

    ones16 = jnp.ones((16,), jnp.float32)

    def _consume(ci, rows_v):
        pltpu.sync_copy(rows_v, acc_sh.at[dst_v.at[ci]], add=True)

        def _deg(j, carry):
            idx = dst_v[ci, pl.ds(j * 16, 16)]
            plsc.addupdate_scatter(deg_v, [idx], ones16)
            return carry
        lax.fori_loop(0, CH // 16, _deg, 0)

    def _gather(ci, rows_v, sem):
        pltpu.async_copy(x_hbm.at[src_v.at[ci]], rows_v, sem)

    def _gather_wait(rows_v, sem):
        pltpu.make_async_copy(x_hbm.at[src_v.at[0]], rows_v, sem).wait()

    def _superblock(sb, carry):
        pltpu.sync_copy(src_hbm.at[wid, sb], src_v)
        pltpu.sync_copy(dst_hbm.at[wid, sb], dst_v)

        _gather(0, rows0_v, sem0)

        def _pair(p, carry2):
            e = p * 2
            _gather_wait(rows0_v, sem0)
            _gather(e + 1, rows1_v, sem1)
            _consume(e, rows0_v)
            _gather_wait(rows1_v, sem1)
            _gather(e + 2, rows0_v, sem0)
            _consume(e + 1, rows1_v)
            return carry2

        lax.fori_loop(0, NPAIR, _pair, carry)
        _gather_wait(rows0_v, sem0)
        _consume(SB - 1, rows0_v)
        return carry

    lax.fori_loop(0, NSB, _superblock, 0)
    plsc.subcore_barrier()

    pltpu.sync_copy(acc_sh.at[pl.ds(slab, RPT)],
                    aggp_hbm.at[c, pl.ds(slab, RPT)])
    pltpu.sync_copy(deg_v, degp_hbm.at[pl.ds(wid * N_PAD, N_PAD)])


def _tc_layer_body(aggp_ref, degp_ref, x_ref, wl_ref, bl_ref, wr_ref, o_ref,
                   *, relu):
    agg = aggp_ref[0] + aggp_ref[1]
    deg = jnp.sum(degp_ref[...], axis=1)
    mean = agg / jnp.maximum(deg, 1.0)[:, None]
    dn = (((1,), (1,)), ((), ()))
    out = lax.dot_general(mean, wl_ref[...], dn,
                          precision=lax.Precision.HIGHEST,
                          preferred_element_type=jnp.float32)
    out = out + bl_ref[...]
    out = out + lax.dot_general(x_ref[...], wr_ref[...], dn,
                                precision=lax.Precision.HIGHEST,
                                preferred_element_type=jnp.float32)
    if relu:
        out = jnp.maximum(out, 0.0)
    o_ref[...] = out


def _tc_layer(aggp, degp_t, x, wl, bl, wr, relu):
    blk = 1024
    grid = (N_PAD // blk,)
    return pl.pallas_call(
        functools.partial(_tc_layer_body, relu=relu),
        grid=grid,
        in_specs=[
            pl.BlockSpec((NC, blk, D), lambda i: (0, i, 0)),
            pl.BlockSpec((blk, NW), lambda i: (i, 0)),
            pl.BlockSpec((blk, D), lambda i: (i, 0)),
            pl.BlockSpec((D, D), lambda i: (0, 0)),
            pl.BlockSpec((1, D), lambda i: (0, 0)),
            pl.BlockSpec((D, D), lambda i: (0, 0)),
        ],
        out_specs=pl.BlockSpec((blk, D), lambda i: (i, 0)),
        out_shape=jax.ShapeDtypeStruct((N_PAD, D), jnp.float32),
    )(aggp, degp_t, x, wl, bl, wr)


def kernel(x, edge_index, W1l, b1l, W1r, W2l, b2l, W2r):
    src = edge_index[0].reshape(NW, NSB, SB, CH)
    dst = edge_index[1].reshape(NW, NSB, SB, CH)
    x_pad = jnp.pad(x, ((0, N_PAD - N_NODES), (0, 0)))

    aggp1, degp = _sc_aggregate(src, dst, x_pad)
    degp_t = degp.reshape(NW, N_PAD).T
    h = _tc_layer(aggp1, degp_t, x_pad, W1l, b1l.reshape(1, D), W1r, relu=True)
    aggp2, _ = _sc_aggregate(src, dst, h)
    out = _tc_layer(aggp2, degp_t, h, W2l, b2l.reshape(1, D), W2r, relu=False)
    return out[:N_NODES]
